# baseline (speedup 1.0000x reference)
import functools

import jax
import jax.numpy as jnp
from jax import lax
from jax.experimental import pallas as pl
from jax.experimental.pallas import tpu as pltpu
from jax.experimental.pallas import tpu_sc as plsc

N = 10000
E = 320000
D_IN = 128
D_H = 64
N_GRAPHS = 64

NC = 2
NS = 16
NW = NC * NS
CH = 128
NB = 8
CPW = 80
E_PAD = NW * CPW * CH

_F32 = jnp.float32
_HIGH = lax.Precision.HIGHEST


def _dot(a, b):
    return lax.dot_general(a, b, (((1,), (0,)), ((), ())),
                           preferred_element_type=_F32, precision=_HIGH)


def _dot_t(a, b):
    return lax.dot_general(a, b, (((0,), (0,)), ((), ())),
                           preferred_element_type=_F32, precision=_HIGH)


_ROWS_PER_SUB = 624
_TAIL = N - NS * _ROWS_PER_SUB


def _sc_agg_body(feat_hbm, src_hbm, dst_hbm, zeros_hbm, out_hbm,
                 sidx_v, didx_v, rows_v, acc_sh, gsem_arr, ssem_arr):
    gsem = [gsem_arr.at[j] for j in range(NB)]
    ssem = [ssem_arr.at[j] for j in range(NB)]
    cid = lax.axis_index("c")
    sid = lax.axis_index("s")
    wid = sid * NC + cid

    with jax.named_scope("agg_init"):
        i1 = pltpu.async_copy(src_hbm.at[pl.ds(wid * CPW, CPW)], sidx_v, gsem[0])
        i2 = pltpu.async_copy(dst_hbm.at[pl.ds(wid * CPW, CPW)], didx_v, gsem[1])

        pltpu.sync_copy(zeros_hbm.at[pl.ds(sid * _ROWS_PER_SUB, _ROWS_PER_SUB)],
                        acc_sh.at[pl.ds(sid * _ROWS_PER_SUB, _ROWS_PER_SUB)])

        @pl.when(sid == NS - 1)
        def _():
            pltpu.sync_copy(zeros_hbm.at[pl.ds(NS * _ROWS_PER_SUB, _TAIL)],
                            acc_sh.at[pl.ds(NS * _ROWS_PER_SUB, _TAIL)])

        i1.wait()
        i2.wait()
        plsc.subcore_barrier()

    with jax.named_scope("agg_loop"):
        @pl.loop(0, CPW // NB)
        def _(t):
            gathers = []
            for j in range(NB):
                c = t * NB + j

                @pl.when(t > 0)
                def _():
                    pltpu.make_async_copy(feat_hbm.at[pl.ds(0, CH)],
                                          rows_v.at[j], ssem[j]).wait()

                gathers.append(pltpu.async_copy(
                    feat_hbm.at[sidx_v.at[c]], rows_v.at[j], gsem[j]))
            for j in range(NB):
                c = t * NB + j
                gathers[j].wait()
                pltpu.async_copy(rows_v.at[j], acc_sh.at[didx_v.at[c]],
                                 ssem[j], add=True)

        for j in range(NB):
            pltpu.make_async_copy(feat_hbm.at[pl.ds(0, CH)],
                                  rows_v.at[j], ssem[j]).wait()

    with jax.named_scope("agg_out"):
        plsc.subcore_barrier()
        pltpu.sync_copy(acc_sh.at[pl.ds(sid * _ROWS_PER_SUB, _ROWS_PER_SUB)],
                        out_hbm.at[cid, pl.ds(sid * _ROWS_PER_SUB, _ROWS_PER_SUB)])

        @pl.when(sid == NS - 1)
        def _():
            pltpu.sync_copy(acc_sh.at[pl.ds(NS * _ROWS_PER_SUB, _TAIL)],
                            out_hbm.at[cid, pl.ds(NS * _ROWS_PER_SUB, _TAIL)])


_sc_agg = pl.kernel(
    _sc_agg_body,
    out_type=jax.ShapeDtypeStruct((NC, N, D_H), _F32),
    mesh=plsc.VectorSubcoreMesh(core_axis_name="c", subcore_axis_name="s"),
    scratch_types=[
        pltpu.VMEM((CPW, CH), jnp.int32),
        pltpu.VMEM((CPW, CH), jnp.int32),
        pltpu.VMEM((NB, CH, D_H), _F32),
        pltpu.VMEM_SHARED((N + 512, D_H), _F32),
        pltpu.SemaphoreType.DMA((NB,)),
        pltpu.SemaphoreType.DMA((NB,)),
    ],
    compiler_params=pltpu.CompilerParams(use_tc_tiling_on_sc=False),
)


def _tc_pre_body(x_ref, wrel_ref, wroot_ref, xr_out, root_out):
    x = x_ref[...]
    xr_out[...] = _dot(x, wrel_ref[...])
    root_out[...] = _dot(x, wroot_ref[...])


def _tc_pre(x, wrel, wroot):
    return pl.pallas_call(
        _tc_pre_body,
        out_shape=(jax.ShapeDtypeStruct((N, D_H), _F32),
                   jax.ShapeDtypeStruct((N, D_H), _F32)),
    )(x, wrel, wroot)


def _tc_mid_body(aggp_ref, root_ref, b_ref, gamma_ref, beta_ref,
                 wrel_ref, wroot_ref, xr_out, root_out):
    pre = aggp_ref[0] + aggp_ref[1] + root_ref[...] + b_ref[...]
    mu = jnp.mean(pre, axis=0, keepdims=True)
    var = jnp.mean((pre - mu) * (pre - mu), axis=0, keepdims=True)
    h = gamma_ref[...] * (pre - mu) * lax.rsqrt(var + 1e-5) + beta_ref[...]
    h = jnp.maximum(h, 0.0)
    xr_out[...] = _dot(h, wrel_ref[...])
    root_out[...] = _dot(h, wroot_ref[...])


def _tc_mid(aggp, root, b, gamma, beta, wrel, wroot):
    return pl.pallas_call(
        _tc_mid_body,
        out_shape=(jax.ShapeDtypeStruct((N, D_H), _F32),
                   jax.ShapeDtypeStruct((N, D_H), _F32)),
    )(aggp, root, b.reshape(1, D_H), gamma.reshape(1, D_H),
      beta.reshape(1, D_H), wrel, wroot)


def _tc_final_body(aggp_ref, root_ref, b_ref, batch_ref, wc1_ref, bc1_ref,
                   wc2_ref, bc2_ref, out_ref):
    h = jnp.maximum(aggp_ref[0] + aggp_ref[1] + root_ref[...] + b_ref[...], 0.0)
    seg = batch_ref[...]
    ids = lax.broadcasted_iota(jnp.int32, (1, N_GRAPHS), 1)
    mask = (seg == ids).astype(_F32)
    s = _dot_t(mask, h)
    cnt = _dot_t(mask, jnp.ones((N, 1), _F32))
    g = s / jnp.maximum(cnt, 1.0)
    g = jnp.maximum(_dot(g, wc1_ref[...]) + bc1_ref[...], 0.0)
    out_ref[...] = _dot(g, wc2_ref[...]) + bc2_ref[...]


def _tc_final(aggp, root, b, batch, wc1, bc1, wc2, bc2):
    return pl.pallas_call(
        _tc_final_body,
        out_shape=jax.ShapeDtypeStruct((N_GRAPHS, 1), _F32),
    )(aggp, root, b.reshape(1, D_H), batch.reshape(N, 1), wc1,
      bc1.reshape(1, D_H), wc2, bc2.reshape(1, 1))


def kernel(x, edge_index, batch, W_rel1, b_rel1, W_root1, gamma1, beta1,
           W_rel2, b_rel2, W_root2, gamma2, beta2,
           W_rel3, b_rel3, W_root3, Wc1, bc1, Wc2, bc2):
    pad = E_PAD - E
    src = jnp.concatenate([edge_index[0], jnp.zeros((pad,), jnp.int32)])
    src = src.reshape(E_PAD // CH, CH)
    dst_pad = N + (jnp.arange(pad, dtype=jnp.int32) % 512)
    dst = jnp.concatenate([edge_index[1], dst_pad])
    dst = dst.reshape(E_PAD // CH, CH)
    zeros = jnp.zeros((N, D_H), _F32)

    xr1, root1 = _tc_pre(x, W_rel1, W_root1)
    agg1 = _sc_agg(xr1, src, dst, zeros)
    xr2, root2 = _tc_mid(agg1, root1, b_rel1, gamma1, beta1, W_rel2, W_root2)
    agg2 = _sc_agg(xr2, src, dst, zeros)
    xr3, root3 = _tc_mid(agg2, root2, b_rel2, gamma2, beta2, W_rel3, W_root3)
    agg3 = _sc_agg(xr3, src, dst, zeros)
    return _tc_final(agg3, root3, b_rel3, batch, Wc1, bc1, Wc2, bc2)


# speedup vs baseline: 2.6778x; 2.6778x over previous
import functools

import jax
import jax.numpy as jnp
from jax import lax
from jax.experimental import pallas as pl
from jax.experimental.pallas import tpu as pltpu
from jax.experimental.pallas import tpu_sc as plsc

N = 10000
E = 320000
D_IN = 128
D_H = 64
N_GRAPHS = 64

NC = 2
NS = 16
NW = NC * NS
CH = 128
NB = 8
CPW = 80
E_PAD = NW * CPW * CH

_F32 = jnp.float32
_HIGH = lax.Precision.HIGHEST


def _dot(a, b):
    return lax.dot_general(a, b, (((1,), (0,)), ((), ())),
                           preferred_element_type=_F32, precision=_HIGH)


def _dot_t(a, b):
    return lax.dot_general(a, b, (((0,), (0,)), ((), ())),
                           preferred_element_type=_F32, precision=_HIGH)


_ROWS_PER_SUB = 624
_TAIL = N - NS * _ROWS_PER_SUB


def _sc_agg_body(feat_hbm, src_hbm, dst_hbm, zeros_hbm, out_hbm,
                 sidx_v, didx_v, rows_v, acc_sh, gsem_arr, ssem_arr):
    gsem = [gsem_arr.at[j] for j in range(NB)]
    ssem = [ssem_arr.at[j] for j in range(NB)]
    cid = lax.axis_index("c")
    sid = lax.axis_index("s")
    wid = sid * NC + cid

    with jax.named_scope("agg_init"):
        i1 = pltpu.async_copy(src_hbm.at[pl.ds(wid * CPW, CPW)], sidx_v, gsem[0])
        i2 = pltpu.async_copy(dst_hbm.at[pl.ds(wid * CPW, CPW)], didx_v, gsem[1])

        pltpu.sync_copy(zeros_hbm.at[pl.ds(sid * _ROWS_PER_SUB, _ROWS_PER_SUB)],
                        acc_sh.at[pl.ds(sid * _ROWS_PER_SUB, _ROWS_PER_SUB)])

        @pl.when(sid == NS - 1)
        def _():
            pltpu.sync_copy(zeros_hbm.at[pl.ds(NS * _ROWS_PER_SUB, _TAIL)],
                            acc_sh.at[pl.ds(NS * _ROWS_PER_SUB, _TAIL)])

        i1.wait()
        i2.wait()
        plsc.subcore_barrier()

    with jax.named_scope("agg_loop"):
        @pl.loop(0, CPW // NB)
        def _(t):
            gathers = []
            for j in range(NB):
                c = t * NB + j

                @pl.when(t > 0)
                def _():
                    pltpu.make_async_copy(feat_hbm.at[pl.ds(0, CH)],
                                          rows_v.at[j], ssem[j]).wait()

                gathers.append(pltpu.async_copy(
                    feat_hbm.at[sidx_v.at[c]], rows_v.at[j], gsem[j]))
            for j in range(NB):
                c = t * NB + j
                gathers[j].wait()
                pltpu.async_copy(rows_v.at[j], acc_sh.at[didx_v.at[c]],
                                 ssem[j], add=True)

        for j in range(NB):
            pltpu.make_async_copy(feat_hbm.at[pl.ds(0, CH)],
                                  rows_v.at[j], ssem[j]).wait()

    with jax.named_scope("agg_out"):
        plsc.subcore_barrier()
        pltpu.sync_copy(acc_sh.at[pl.ds(sid * _ROWS_PER_SUB, _ROWS_PER_SUB)],
                        out_hbm.at[cid, pl.ds(sid * _ROWS_PER_SUB, _ROWS_PER_SUB)])

        @pl.when(sid == NS - 1)
        def _():
            pltpu.sync_copy(acc_sh.at[pl.ds(NS * _ROWS_PER_SUB, _TAIL)],
                            out_hbm.at[cid, pl.ds(NS * _ROWS_PER_SUB, _TAIL)])


_sc_agg = pl.kernel(
    _sc_agg_body,
    out_type=jax.ShapeDtypeStruct((NC, N, D_H), _F32),
    mesh=plsc.VectorSubcoreMesh(core_axis_name="c", subcore_axis_name="s"),
    scratch_types=[
        pltpu.VMEM((CPW, CH), jnp.int32),
        pltpu.VMEM((CPW, CH), jnp.int32),
        pltpu.VMEM((NB, CH, D_H), _F32),
        pltpu.VMEM_SHARED((N + 512, D_H), _F32),
        pltpu.SemaphoreType.DMA((NB,)),
        pltpu.SemaphoreType.DMA((NB,)),
    ],
    compiler_params=pltpu.CompilerParams(use_tc_tiling_on_sc=False),
)


def _tc_pre_body(x_ref, wrel_ref, wroot_ref, xr_out, root_out):
    x = x_ref[...]
    xr_out[...] = _dot(x, wrel_ref[...])
    root_out[...] = _dot(x, wroot_ref[...])


def _tc_pre(x, wrel, wroot):
    return pl.pallas_call(
        _tc_pre_body,
        out_shape=(jax.ShapeDtypeStruct((N, D_H), _F32),
                   jax.ShapeDtypeStruct((N, D_H), _F32)),
    )(x, wrel, wroot)


def _tc_mid_body(aggp_ref, root_ref, b_ref, gamma_ref, beta_ref,
                 wrel_ref, wroot_ref, xr_out, root_out):
    pre = aggp_ref[0] + aggp_ref[1] + root_ref[...] + b_ref[...]
    mu = jnp.mean(pre, axis=0, keepdims=True)
    var = jnp.mean((pre - mu) * (pre - mu), axis=0, keepdims=True)
    h = gamma_ref[...] * (pre - mu) * lax.rsqrt(var + 1e-5) + beta_ref[...]
    h = jnp.maximum(h, 0.0)
    xr_out[...] = _dot(h, wrel_ref[...])
    root_out[...] = _dot(h, wroot_ref[...])


def _tc_mid(aggp, root, b, gamma, beta, wrel, wroot):
    return pl.pallas_call(
        _tc_mid_body,
        out_shape=(jax.ShapeDtypeStruct((N, D_H), _F32),
                   jax.ShapeDtypeStruct((N, D_H), _F32)),
    )(aggp, root, b.reshape(1, D_H), gamma.reshape(1, D_H),
      beta.reshape(1, D_H), wrel, wroot)


def _tc_final_body(aggp_ref, root_ref, b_ref, batch_ref, wc1_ref, bc1_ref,
                   wc2_ref, bc2_ref, out_ref):
    h = jnp.maximum(aggp_ref[0] + aggp_ref[1] + root_ref[...] + b_ref[...], 0.0)
    seg = batch_ref[...]
    ids = lax.broadcasted_iota(jnp.int32, (1, N_GRAPHS), 1)
    mask = (seg == ids).astype(_F32)
    s = _dot_t(mask, h)
    cnt = _dot_t(mask, jnp.ones((N, 1), _F32))
    g = s / jnp.maximum(cnt, 1.0)
    g = jnp.maximum(_dot(g, wc1_ref[...]) + bc1_ref[...], 0.0)
    out_ref[...] = _dot(g, wc2_ref[...]) + bc2_ref[...]


def _tc_final(aggp, root, b, batch, wc1, bc1, wc2, bc2):
    return pl.pallas_call(
        _tc_final_body,
        out_shape=jax.ShapeDtypeStruct((N_GRAPHS, 1), _F32),
    )(aggp, root, b.reshape(1, D_H), batch.reshape(N, 1), wc1,
      bc1.reshape(1, D_H), wc2, bc2.reshape(1, 1))


def kernel(x, edge_index, batch, W_rel1, b_rel1, W_root1, gamma1, beta1,
           W_rel2, b_rel2, W_root2, gamma2, beta2,
           W_rel3, b_rel3, W_root3, Wc1, bc1, Wc2, bc2):
    pad = E_PAD - E
    src_pad = jnp.arange(pad, dtype=jnp.int32) * 13 % N
    src = jnp.concatenate([edge_index[0], src_pad])
    src = src.reshape(E_PAD // CH, CH)
    dst_pad = N + (jnp.arange(pad, dtype=jnp.int32) % 512)
    dst = jnp.concatenate([edge_index[1], dst_pad])
    dst = dst.reshape(E_PAD // CH, CH)
    zeros = jnp.zeros((N, D_H), _F32)

    xr1, root1 = _tc_pre(x, W_rel1, W_root1)
    agg1 = _sc_agg(xr1, src, dst, zeros)
    xr2, root2 = _tc_mid(agg1, root1, b_rel1, gamma1, beta1, W_rel2, W_root2)
    agg2 = _sc_agg(xr2, src, dst, zeros)
    xr3, root3 = _tc_mid(agg2, root2, b_rel2, gamma2, beta2, W_rel3, W_root3)
    agg3 = _sc_agg(xr3, src, dst, zeros)
    return _tc_final(agg3, root3, b_rel3, batch, Wc1, bc1, Wc2, bc2)


# speedup vs baseline: 2.7029x; 1.0094x over previous
import functools

import jax
import jax.numpy as jnp
from jax import lax
from jax.experimental import pallas as pl
from jax.experimental.pallas import tpu as pltpu
from jax.experimental.pallas import tpu_sc as plsc

N = 10000
E = 320000
D_IN = 128
D_H = 64
N_GRAPHS = 64

NC = 2
NS = 16
NW = NC * NS
CH = 128
NB = 8
CHUNKS = E // CH
CPW = CHUNKS // NW
XTRA = CHUNKS - CPW * NW
FULL_T = CPW // NB
TAILC = CPW - FULL_T * NB

_F32 = jnp.float32
_HIGH = lax.Precision.HIGHEST


def _dot(a, b):
    return lax.dot_general(a, b, (((1,), (0,)), ((), ())),
                           preferred_element_type=_F32, precision=_HIGH)


def _dot_t(a, b):
    return lax.dot_general(a, b, (((0,), (0,)), ((), ())),
                           preferred_element_type=_F32, precision=_HIGH)


_ROWS_PER_SUB = 624
_TAIL = N - NS * _ROWS_PER_SUB


def _sc_agg_body(feat_hbm, src_hbm, dst_hbm, zeros_hbm, out_hbm,
                 sidx_v, didx_v, rows_v, acc_sh, gsem_arr, ssem_arr):
    gsem = [gsem_arr.at[j] for j in range(NB)]
    ssem = [ssem_arr.at[j] for j in range(NB)]
    cid = lax.axis_index("c")
    sid = lax.axis_index("s")
    wid = sid * NC + cid
    has_extra = wid < XTRA
    start = wid * CPW + lax.min(wid, XTRA)

    def _drain_scatter(j):
        pltpu.make_async_copy(feat_hbm.at[pl.ds(0, CH)],
                              rows_v.at[j], ssem[j]).wait()

    with jax.named_scope("agg_init"):
        i1 = pltpu.async_copy(src_hbm.at[pl.ds(start, CPW)],
                              sidx_v.at[pl.ds(0, CPW)], gsem[0])
        i2 = pltpu.async_copy(dst_hbm.at[pl.ds(start, CPW)],
                              didx_v.at[pl.ds(0, CPW)], gsem[1])

        @pl.when(has_extra)
        def _():
            pltpu.sync_copy(src_hbm.at[pl.ds(start + CPW, 1)],
                            sidx_v.at[pl.ds(CPW, 1)])
            pltpu.sync_copy(dst_hbm.at[pl.ds(start + CPW, 1)],
                            didx_v.at[pl.ds(CPW, 1)])

        pltpu.sync_copy(zeros_hbm.at[pl.ds(sid * _ROWS_PER_SUB, _ROWS_PER_SUB)],
                        acc_sh.at[pl.ds(sid * _ROWS_PER_SUB, _ROWS_PER_SUB)])

        @pl.when(sid == NS - 1)
        def _():
            pltpu.sync_copy(zeros_hbm.at[pl.ds(NS * _ROWS_PER_SUB, _TAIL)],
                            acc_sh.at[pl.ds(NS * _ROWS_PER_SUB, _TAIL)])

        i1.wait()
        i2.wait()
        plsc.subcore_barrier()

    with jax.named_scope("agg_loop"):
        @pl.loop(0, FULL_T)
        def _(t):
            gathers = []
            for j in range(NB):
                c = t * NB + j

                @pl.when(t > 0)
                def _():
                    _drain_scatter(j)

                gathers.append(pltpu.async_copy(
                    feat_hbm.at[sidx_v.at[c]], rows_v.at[j], gsem[j]))
            for j in range(NB):
                c = t * NB + j
                gathers[j].wait()
                pltpu.async_copy(rows_v.at[j], acc_sh.at[didx_v.at[c]],
                                 ssem[j], add=True)

        tail_gathers = []
        for j in range(TAILC):
            c = FULL_T * NB + j
            _drain_scatter(j)
            tail_gathers.append(pltpu.async_copy(
                feat_hbm.at[sidx_v.at[c]], rows_v.at[j], gsem[j]))
        for j in range(TAILC):
            c = FULL_T * NB + j
            tail_gathers[j].wait()
            pltpu.async_copy(rows_v.at[j], acc_sh.at[didx_v.at[c]],
                             ssem[j], add=True)

        @pl.when(has_extra)
        def _():
            _drain_scatter(TAILC)
            g = pltpu.async_copy(feat_hbm.at[sidx_v.at[CPW]],
                                 rows_v.at[TAILC], gsem[TAILC])
            g.wait()
            pltpu.async_copy(rows_v.at[TAILC], acc_sh.at[didx_v.at[CPW]],
                             ssem[TAILC], add=True)

        for j in range(NB):
            _drain_scatter(j)

    with jax.named_scope("agg_out"):
        plsc.subcore_barrier()
        pltpu.sync_copy(acc_sh.at[pl.ds(sid * _ROWS_PER_SUB, _ROWS_PER_SUB)],
                        out_hbm.at[cid, pl.ds(sid * _ROWS_PER_SUB, _ROWS_PER_SUB)])

        @pl.when(sid == NS - 1)
        def _():
            pltpu.sync_copy(acc_sh.at[pl.ds(NS * _ROWS_PER_SUB, _TAIL)],
                            out_hbm.at[cid, pl.ds(NS * _ROWS_PER_SUB, _TAIL)])


_sc_agg = pl.kernel(
    _sc_agg_body,
    out_type=jax.ShapeDtypeStruct((NC, N, D_H), _F32),
    mesh=plsc.VectorSubcoreMesh(core_axis_name="c", subcore_axis_name="s"),
    scratch_types=[
        pltpu.VMEM((CPW + 1, CH), jnp.int32),
        pltpu.VMEM((CPW + 1, CH), jnp.int32),
        pltpu.VMEM((NB, CH, D_H), _F32),
        pltpu.VMEM_SHARED((N, D_H), _F32),
        pltpu.SemaphoreType.DMA((NB,)),
        pltpu.SemaphoreType.DMA((NB,)),
    ],
    compiler_params=pltpu.CompilerParams(use_tc_tiling_on_sc=False),
)


def _tc_pre_body(x_ref, wrel_ref, wroot_ref, xr_out, root_out):
    x = x_ref[...]
    xr_out[...] = _dot(x, wrel_ref[...])
    root_out[...] = _dot(x, wroot_ref[...])


def _tc_pre(x, wrel, wroot):
    return pl.pallas_call(
        _tc_pre_body,
        out_shape=(jax.ShapeDtypeStruct((N, D_H), _F32),
                   jax.ShapeDtypeStruct((N, D_H), _F32)),
    )(x, wrel, wroot)


def _tc_mid_body(aggp_ref, root_ref, b_ref, gamma_ref, beta_ref,
                 wrel_ref, wroot_ref, xr_out, root_out):
    pre = aggp_ref[0] + aggp_ref[1] + root_ref[...] + b_ref[...]
    mu = jnp.mean(pre, axis=0, keepdims=True)
    var = jnp.mean((pre - mu) * (pre - mu), axis=0, keepdims=True)
    h = gamma_ref[...] * (pre - mu) * lax.rsqrt(var + 1e-5) + beta_ref[...]
    h = jnp.maximum(h, 0.0)
    xr_out[...] = _dot(h, wrel_ref[...])
    root_out[...] = _dot(h, wroot_ref[...])


def _tc_mid(aggp, root, b, gamma, beta, wrel, wroot):
    return pl.pallas_call(
        _tc_mid_body,
        out_shape=(jax.ShapeDtypeStruct((N, D_H), _F32),
                   jax.ShapeDtypeStruct((N, D_H), _F32)),
    )(aggp, root, b.reshape(1, D_H), gamma.reshape(1, D_H),
      beta.reshape(1, D_H), wrel, wroot)


def _tc_final_body(aggp_ref, root_ref, b_ref, batch_ref, wc1_ref, bc1_ref,
                   wc2_ref, bc2_ref, out_ref):
    h = jnp.maximum(aggp_ref[0] + aggp_ref[1] + root_ref[...] + b_ref[...], 0.0)
    seg = batch_ref[...]
    ids = lax.broadcasted_iota(jnp.int32, (1, N_GRAPHS), 1)
    mask = (seg == ids).astype(_F32)
    s = _dot_t(mask, h)
    cnt = _dot_t(mask, jnp.ones((N, 1), _F32))
    g = s / jnp.maximum(cnt, 1.0)
    g = jnp.maximum(_dot(g, wc1_ref[...]) + bc1_ref[...], 0.0)
    out_ref[...] = _dot(g, wc2_ref[...]) + bc2_ref[...]


def _tc_final(aggp, root, b, batch, wc1, bc1, wc2, bc2):
    return pl.pallas_call(
        _tc_final_body,
        out_shape=jax.ShapeDtypeStruct((N_GRAPHS, 1), _F32),
    )(aggp, root, b.reshape(1, D_H), batch.reshape(N, 1), wc1,
      bc1.reshape(1, D_H), wc2, bc2.reshape(1, 1))


def kernel(x, edge_index, batch, W_rel1, b_rel1, W_root1, gamma1, beta1,
           W_rel2, b_rel2, W_root2, gamma2, beta2,
           W_rel3, b_rel3, W_root3, Wc1, bc1, Wc2, bc2):
    src = edge_index[0].reshape(CHUNKS, CH)
    dst = edge_index[1].reshape(CHUNKS, CH)
    zeros = jnp.zeros((N, D_H), _F32)

    xr1, root1 = _tc_pre(x, W_rel1, W_root1)
    agg1 = _sc_agg(xr1, src, dst, zeros)
    xr2, root2 = _tc_mid(agg1, root1, b_rel1, gamma1, beta1, W_rel2, W_root2)
    agg2 = _sc_agg(xr2, src, dst, zeros)
    xr3, root3 = _tc_mid(agg2, root2, b_rel2, gamma2, beta2, W_rel3, W_root3)
    agg3 = _sc_agg(xr3, src, dst, zeros)
    return _tc_final(agg3, root3, b_rel3, batch, Wc1, bc1, Wc2, bc2)


# speedup vs baseline: 2.9487x; 1.0909x over previous
import functools

import jax
import jax.numpy as jnp
from jax import lax
from jax.experimental import pallas as pl
from jax.experimental.pallas import tpu as pltpu
from jax.experimental.pallas import tpu_sc as plsc

N = 10000
E = 320000
D_IN = 128
D_H = 64
N_GRAPHS = 64

NC = 2
NS = 16
NW = NC * NS
CH = 128
NB = 8
CHUNKS = E // CH
CPW = CHUNKS // NW
XTRA = CHUNKS - CPW * NW
FULL_T = CPW // NB
TAILC = CPW - FULL_T * NB

_F32 = jnp.float32
_HIGH = lax.Precision.DEFAULT


def _dot(a, b):
    return lax.dot_general(a, b, (((1,), (0,)), ((), ())),
                           preferred_element_type=_F32, precision=_HIGH)


def _dot_t(a, b):
    return lax.dot_general(a, b, (((0,), (0,)), ((), ())),
                           preferred_element_type=_F32, precision=_HIGH)


_ROWS_PER_SUB = 624
_TAIL = N - NS * _ROWS_PER_SUB


def _sc_agg_body(feat_hbm, src_hbm, dst_hbm, zeros_hbm, out_hbm,
                 sidx_v, didx_v, rows_v, acc_sh, gsem_arr, ssem_arr):
    gsem = [gsem_arr.at[j] for j in range(NB)]
    ssem = [ssem_arr.at[j] for j in range(NB)]
    cid = lax.axis_index("c")
    sid = lax.axis_index("s")
    wid = sid * NC + cid
    has_extra = wid < XTRA
    start = wid * CPW + lax.min(wid, XTRA)

    def _drain_scatter(j):
        pltpu.make_async_copy(feat_hbm.at[pl.ds(0, CH)],
                              rows_v.at[j], ssem[j]).wait()

    with jax.named_scope("agg_init"):
        i1 = pltpu.async_copy(src_hbm.at[pl.ds(start, CPW)],
                              sidx_v.at[pl.ds(0, CPW)], gsem[0])
        i2 = pltpu.async_copy(dst_hbm.at[pl.ds(start, CPW)],
                              didx_v.at[pl.ds(0, CPW)], gsem[1])

        @pl.when(has_extra)
        def _():
            pltpu.sync_copy(src_hbm.at[pl.ds(start + CPW, 1)],
                            sidx_v.at[pl.ds(CPW, 1)])
            pltpu.sync_copy(dst_hbm.at[pl.ds(start + CPW, 1)],
                            didx_v.at[pl.ds(CPW, 1)])

        pltpu.sync_copy(zeros_hbm.at[pl.ds(sid * _ROWS_PER_SUB, _ROWS_PER_SUB)],
                        acc_sh.at[pl.ds(sid * _ROWS_PER_SUB, _ROWS_PER_SUB)])

        @pl.when(sid == NS - 1)
        def _():
            pltpu.sync_copy(zeros_hbm.at[pl.ds(NS * _ROWS_PER_SUB, _TAIL)],
                            acc_sh.at[pl.ds(NS * _ROWS_PER_SUB, _TAIL)])

        i1.wait()
        i2.wait()
        plsc.subcore_barrier()

    with jax.named_scope("agg_loop"):
        @pl.loop(0, FULL_T)
        def _(t):
            gathers = []
            for j in range(NB):
                c = t * NB + j

                @pl.when(t > 0)
                def _():
                    _drain_scatter(j)

                gathers.append(pltpu.async_copy(
                    feat_hbm.at[sidx_v.at[c]], rows_v.at[j], gsem[j]))
            for j in range(NB):
                c = t * NB + j
                gathers[j].wait()
                pltpu.async_copy(rows_v.at[j], acc_sh.at[didx_v.at[c]],
                                 ssem[j], add=True)

        tail_gathers = []
        for j in range(TAILC):
            c = FULL_T * NB + j
            _drain_scatter(j)
            tail_gathers.append(pltpu.async_copy(
                feat_hbm.at[sidx_v.at[c]], rows_v.at[j], gsem[j]))
        for j in range(TAILC):
            c = FULL_T * NB + j
            tail_gathers[j].wait()
            pltpu.async_copy(rows_v.at[j], acc_sh.at[didx_v.at[c]],
                             ssem[j], add=True)

        @pl.when(has_extra)
        def _():
            _drain_scatter(TAILC)
            g = pltpu.async_copy(feat_hbm.at[sidx_v.at[CPW]],
                                 rows_v.at[TAILC], gsem[TAILC])
            g.wait()
            pltpu.async_copy(rows_v.at[TAILC], acc_sh.at[didx_v.at[CPW]],
                             ssem[TAILC], add=True)

        for j in range(NB):
            _drain_scatter(j)

    with jax.named_scope("agg_out"):
        plsc.subcore_barrier()
        pltpu.sync_copy(acc_sh.at[pl.ds(sid * _ROWS_PER_SUB, _ROWS_PER_SUB)],
                        out_hbm.at[cid, pl.ds(sid * _ROWS_PER_SUB, _ROWS_PER_SUB)])

        @pl.when(sid == NS - 1)
        def _():
            pltpu.sync_copy(acc_sh.at[pl.ds(NS * _ROWS_PER_SUB, _TAIL)],
                            out_hbm.at[cid, pl.ds(NS * _ROWS_PER_SUB, _TAIL)])


_sc_agg = pl.kernel(
    _sc_agg_body,
    out_type=jax.ShapeDtypeStruct((NC, N, D_H), _F32),
    mesh=plsc.VectorSubcoreMesh(core_axis_name="c", subcore_axis_name="s"),
    scratch_types=[
        pltpu.VMEM((CPW + 1, CH), jnp.int32),
        pltpu.VMEM((CPW + 1, CH), jnp.int32),
        pltpu.VMEM((NB, CH, D_H), _F32),
        pltpu.VMEM_SHARED((N, D_H), _F32),
        pltpu.SemaphoreType.DMA((NB,)),
        pltpu.SemaphoreType.DMA((NB,)),
    ],
    compiler_params=pltpu.CompilerParams(use_tc_tiling_on_sc=False),
)


def _tc_pre_body(x_ref, wrel_ref, wroot_ref, xr_out, root_out):
    x = x_ref[...]
    xr_out[...] = _dot(x, wrel_ref[...])
    root_out[...] = _dot(x, wroot_ref[...])


def _tc_pre(x, wrel, wroot):
    return pl.pallas_call(
        _tc_pre_body,
        out_shape=(jax.ShapeDtypeStruct((N, D_H), _F32),
                   jax.ShapeDtypeStruct((N, D_H), _F32)),
    )(x, wrel, wroot)


def _tc_mid_body(aggp_ref, root_ref, b_ref, gamma_ref, beta_ref,
                 wrel_ref, wroot_ref, xr_out, root_out):
    pre = aggp_ref[0] + aggp_ref[1] + root_ref[...] + b_ref[...]
    mu = jnp.mean(pre, axis=0, keepdims=True)
    var = jnp.mean((pre - mu) * (pre - mu), axis=0, keepdims=True)
    h = gamma_ref[...] * (pre - mu) * lax.rsqrt(var + 1e-5) + beta_ref[...]
    h = jnp.maximum(h, 0.0)
    xr_out[...] = _dot(h, wrel_ref[...])
    root_out[...] = _dot(h, wroot_ref[...])


def _tc_mid(aggp, root, b, gamma, beta, wrel, wroot):
    return pl.pallas_call(
        _tc_mid_body,
        out_shape=(jax.ShapeDtypeStruct((N, D_H), _F32),
                   jax.ShapeDtypeStruct((N, D_H), _F32)),
    )(aggp, root, b.reshape(1, D_H), gamma.reshape(1, D_H),
      beta.reshape(1, D_H), wrel, wroot)


def _tc_final_body(aggp_ref, root_ref, b_ref, batch_ref, wc1_ref, bc1_ref,
                   wc2_ref, bc2_ref, out_ref):
    h = jnp.maximum(aggp_ref[0] + aggp_ref[1] + root_ref[...] + b_ref[...], 0.0)
    seg = batch_ref[...]
    ids = lax.broadcasted_iota(jnp.int32, (1, N_GRAPHS), 1)
    mask = (seg == ids).astype(_F32)
    s = _dot_t(mask, h)
    cnt = _dot_t(mask, jnp.ones((N, 1), _F32))
    g = s / jnp.maximum(cnt, 1.0)
    g = jnp.maximum(_dot(g, wc1_ref[...]) + bc1_ref[...], 0.0)
    out_ref[...] = _dot(g, wc2_ref[...]) + bc2_ref[...]


def _tc_final(aggp, root, b, batch, wc1, bc1, wc2, bc2):
    return pl.pallas_call(
        _tc_final_body,
        out_shape=jax.ShapeDtypeStruct((N_GRAPHS, 1), _F32),
    )(aggp, root, b.reshape(1, D_H), batch.reshape(N, 1), wc1,
      bc1.reshape(1, D_H), wc2, bc2.reshape(1, 1))


def kernel(x, edge_index, batch, W_rel1, b_rel1, W_root1, gamma1, beta1,
           W_rel2, b_rel2, W_root2, gamma2, beta2,
           W_rel3, b_rel3, W_root3, Wc1, bc1, Wc2, bc2):
    src = edge_index[0].reshape(CHUNKS, CH)
    dst = edge_index[1].reshape(CHUNKS, CH)
    zeros = jnp.zeros((N, D_H), _F32)

    xr1, root1 = _tc_pre(x, W_rel1, W_root1)
    agg1 = _sc_agg(xr1, src, dst, zeros)
    xr2, root2 = _tc_mid(agg1, root1, b_rel1, gamma1, beta1, W_rel2, W_root2)
    agg2 = _sc_agg(xr2, src, dst, zeros)
    xr3, root3 = _tc_mid(agg2, root2, b_rel2, gamma2, beta2, W_rel3, W_root3)
    agg3 = _sc_agg(xr3, src, dst, zeros)
    return _tc_final(agg3, root3, b_rel3, batch, Wc1, bc1, Wc2, bc2)


# speedup vs baseline: 3.5891x; 1.2172x over previous
import functools

import jax
import jax.numpy as jnp
from jax import lax
from jax.experimental import pallas as pl
from jax.experimental.pallas import tpu as pltpu
from jax.experimental.pallas import tpu_sc as plsc

N = 10000
E = 320000
D_IN = 128
D_H = 64
N_GRAPHS = 64

NC = 2
NS = 16
NW = NC * NS
CH = 128
NB = 8
CHUNKS = E // CH
CPW = CHUNKS // NW
XTRA = CHUNKS - CPW * NW
FULL_T = CPW // NB
TAILC = CPW - FULL_T * NB

_F32 = jnp.float32
_HIGH = lax.Precision.DEFAULT


def _dot(a, b):
    return lax.dot_general(a, b, (((1,), (0,)), ((), ())),
                           preferred_element_type=_F32, precision=_HIGH)


def _dot_t(a, b):
    return lax.dot_general(a, b, (((0,), (0,)), ((), ())),
                           preferred_element_type=_F32, precision=_HIGH)


_ROWS_PER_SUB = 624
_TAIL = N - NS * _ROWS_PER_SUB


def _sc_agg_body(feat_hbm, src_hbm, dst_hbm, zeros_hbm, out0_hbm, out1_hbm,
                 sidx_v, didx_v, rows_v, acc_sh, gsem_arr, ssem_arr):
    gsem = [gsem_arr.at[j] for j in range(NB)]
    ssem = [ssem_arr.at[j] for j in range(NB)]
    cid = lax.axis_index("c")
    sid = lax.axis_index("s")
    wid = sid * NC + cid
    has_extra = wid < XTRA
    start = wid * CPW + lax.min(wid, XTRA)

    def _drain_scatter(j):
        pltpu.make_async_copy(feat_hbm.at[pl.ds(0, CH)],
                              rows_v.at[j], ssem[j]).wait()

    with jax.named_scope("agg_init"):
        i1 = pltpu.async_copy(src_hbm.at[pl.ds(start, CPW)],
                              sidx_v.at[pl.ds(0, CPW)], gsem[0])
        i2 = pltpu.async_copy(dst_hbm.at[pl.ds(start, CPW)],
                              didx_v.at[pl.ds(0, CPW)], gsem[1])

        @pl.when(has_extra)
        def _():
            pltpu.sync_copy(src_hbm.at[pl.ds(start + CPW, 1)],
                            sidx_v.at[pl.ds(CPW, 1)])
            pltpu.sync_copy(dst_hbm.at[pl.ds(start + CPW, 1)],
                            didx_v.at[pl.ds(CPW, 1)])

        pltpu.sync_copy(zeros_hbm.at[pl.ds(sid * _ROWS_PER_SUB, _ROWS_PER_SUB)],
                        acc_sh.at[pl.ds(sid * _ROWS_PER_SUB, _ROWS_PER_SUB)])

        @pl.when(sid == NS - 1)
        def _():
            pltpu.sync_copy(zeros_hbm.at[pl.ds(NS * _ROWS_PER_SUB, _TAIL)],
                            acc_sh.at[pl.ds(NS * _ROWS_PER_SUB, _TAIL)])

        i1.wait()
        i2.wait()
        plsc.subcore_barrier()

    with jax.named_scope("agg_loop"):
        @pl.loop(0, FULL_T)
        def _(t):
            gathers = []
            for j in range(NB):
                c = t * NB + j

                @pl.when(t > 0)
                def _():
                    _drain_scatter(j)

                gathers.append(pltpu.async_copy(
                    feat_hbm.at[sidx_v.at[c]], rows_v.at[j], gsem[j]))
            for j in range(NB):
                c = t * NB + j
                gathers[j].wait()
                pltpu.async_copy(rows_v.at[j], acc_sh.at[didx_v.at[c]],
                                 ssem[j], add=True)

        tail_gathers = []
        for j in range(TAILC):
            c = FULL_T * NB + j
            _drain_scatter(j)
            tail_gathers.append(pltpu.async_copy(
                feat_hbm.at[sidx_v.at[c]], rows_v.at[j], gsem[j]))
        for j in range(TAILC):
            c = FULL_T * NB + j
            tail_gathers[j].wait()
            pltpu.async_copy(rows_v.at[j], acc_sh.at[didx_v.at[c]],
                             ssem[j], add=True)

        @pl.when(has_extra)
        def _():
            _drain_scatter(TAILC)
            g = pltpu.async_copy(feat_hbm.at[sidx_v.at[CPW]],
                                 rows_v.at[TAILC], gsem[TAILC])
            g.wait()
            pltpu.async_copy(rows_v.at[TAILC], acc_sh.at[didx_v.at[CPW]],
                             ssem[TAILC], add=True)

        for j in range(NB):
            _drain_scatter(j)

    with jax.named_scope("agg_out"):
        plsc.subcore_barrier()
        for c, out_hbm in ((0, out0_hbm), (1, out1_hbm)):
            @pl.when(cid == c)
            def _():
                pltpu.sync_copy(
                    acc_sh.at[pl.ds(sid * _ROWS_PER_SUB, _ROWS_PER_SUB)],
                    out_hbm.at[pl.ds(sid * _ROWS_PER_SUB, _ROWS_PER_SUB)])

            @pl.when(jnp.logical_and(cid == c, sid == NS - 1))
            def _():
                pltpu.sync_copy(acc_sh.at[pl.ds(NS * _ROWS_PER_SUB, _TAIL)],
                                out_hbm.at[pl.ds(NS * _ROWS_PER_SUB, _TAIL)])


_sc_agg = pl.kernel(
    _sc_agg_body,
    out_type=(jax.ShapeDtypeStruct((N, D_H), _F32),
              jax.ShapeDtypeStruct((N, D_H), _F32)),
    mesh=plsc.VectorSubcoreMesh(core_axis_name="c", subcore_axis_name="s"),
    scratch_types=[
        pltpu.VMEM((CPW + 1, CH), jnp.int32),
        pltpu.VMEM((CPW + 1, CH), jnp.int32),
        pltpu.VMEM((NB, CH, D_H), _F32),
        pltpu.VMEM_SHARED((N, D_H), _F32),
        pltpu.SemaphoreType.DMA((NB,)),
        pltpu.SemaphoreType.DMA((NB,)),
    ],
    compiler_params=pltpu.CompilerParams(use_tc_tiling_on_sc=False),
)


NP = N // 2
DP = 2 * D_H


def _halves_sum(v):
    return v[:, :D_H] + v[:, D_H:]


def _tile2(v):
    return jnp.concatenate([v, v], axis=1)


def _tc_pre_body(x_ref, wrel_ref, wroot_ref, xr_out, root_out):
    x = x_ref[...]
    xr_out[...] = _dot(x, wrel_ref[...])
    root_out[...] = _dot(x, wroot_ref[...])


def _tc_pre(x, wrel, wroot):
    return pl.pallas_call(
        _tc_pre_body,
        out_shape=(jax.ShapeDtypeStruct((NP, DP), _F32),
                   jax.ShapeDtypeStruct((NP, DP), _F32)),
    )(x, wrel, wroot)


def _bn_relu(pre, gamma2, beta2):

    mu = _tile2(_halves_sum(jnp.sum(pre, axis=0, keepdims=True)) / N)
    d = pre - mu
    var = _tile2(_halves_sum(jnp.sum(d * d, axis=0, keepdims=True)) / N)
    h = gamma2 * d * lax.rsqrt(var + 1e-5) + beta2
    return jnp.maximum(h, 0.0)


def _tc_mid_body(agg0_ref, agg1_ref, root_ref, b_ref, gamma_ref, beta_ref,
                 wrel_ref, wroot_ref, xr_out, root_out):
    pre = agg0_ref[...] + agg1_ref[...] + root_ref[...] + b_ref[...]
    h = _bn_relu(pre, gamma_ref[...], beta_ref[...])
    xr_out[...] = _dot(h, wrel_ref[...])
    root_out[...] = _dot(h, wroot_ref[...])


def _tc_mid(agg0, agg1, root, b2, gamma2, beta2, wrel_bd, wroot_bd):
    return pl.pallas_call(
        _tc_mid_body,
        out_shape=(jax.ShapeDtypeStruct((NP, DP), _F32),
                   jax.ShapeDtypeStruct((NP, DP), _F32)),
    )(agg0, agg1, root, b2, gamma2, beta2, wrel_bd, wroot_bd)


def _tc_final_body(agg0_ref, agg1_ref, root_ref, b_ref, batch_ref,
                   wc1_ref, bc1_ref, wc2_ref, bc2_ref, out_ref):
    h = jnp.maximum(agg0_ref[...] + agg1_ref[...] + root_ref[...] + b_ref[...],
                    0.0)
    seg = batch_ref[...]
    ids = lax.broadcasted_iota(jnp.int32, (1, N_GRAPHS), 1)
    mask_a = (seg[:, 0:1] == ids).astype(_F32)
    mask_b = (seg[:, 1:2] == ids).astype(_F32)
    s = _dot_t(mask_a, h[:, :D_H]) + _dot_t(mask_b, h[:, D_H:])
    ones = jnp.ones((NP, 1), _F32)
    cnt = _dot_t(mask_a, ones) + _dot_t(mask_b, ones)
    g = s / jnp.maximum(cnt, 1.0)
    g = jnp.maximum(_dot(g, wc1_ref[...]) + bc1_ref[...], 0.0)
    out_ref[...] = _dot(g, wc2_ref[...]) + bc2_ref[...]


def _tc_final(agg0, agg1, root, b2, batch2, wc1, bc1, wc2, bc2):
    return pl.pallas_call(
        _tc_final_body,
        out_shape=jax.ShapeDtypeStruct((N_GRAPHS, 1), _F32),
    )(agg0, agg1, root, b2, batch2, wc1,
      bc1.reshape(1, D_H), wc2, bc2.reshape(1, 1))


def _blockdiag(w):
    z = jnp.zeros(w.shape, _F32)
    return jnp.concatenate([jnp.concatenate([w, z], axis=1),
                            jnp.concatenate([z, w], axis=1)], axis=0)


def kernel(x, edge_index, batch, W_rel1, b_rel1, W_root1, gamma1, beta1,
           W_rel2, b_rel2, W_root2, gamma2, beta2,
           W_rel3, b_rel3, W_root3, Wc1, bc1, Wc2, bc2):
    src = edge_index[0].reshape(CHUNKS, CH)
    dst = edge_index[1].reshape(CHUNKS, CH)
    zeros = jnp.zeros((N, D_H), _F32)
    t2 = lambda v: jnp.concatenate([v, v]).reshape(1, DP)

    xr1, root1 = _tc_pre(x.reshape(NP, 2 * D_IN),
                         _blockdiag(W_rel1), _blockdiag(W_root1))
    a0, a1 = _sc_agg(xr1.reshape(N, D_H), src, dst, zeros)
    xr2, root2 = _tc_mid(a0.reshape(NP, DP), a1.reshape(NP, DP), root1,
                         t2(b_rel1), t2(gamma1), t2(beta1),
                         _blockdiag(W_rel2), _blockdiag(W_root2))
    a0, a1 = _sc_agg(xr2.reshape(N, D_H), src, dst, zeros)
    xr3, root3 = _tc_mid(a0.reshape(NP, DP), a1.reshape(NP, DP), root2,
                         t2(b_rel2), t2(gamma2), t2(beta2),
                         _blockdiag(W_rel3), _blockdiag(W_root3))
    a0, a1 = _sc_agg(xr3.reshape(N, D_H), src, dst, zeros)
    return _tc_final(a0.reshape(NP, DP), a1.reshape(NP, DP), root3,
                     t2(b_rel3), batch.reshape(NP, 2), Wc1, bc1, Wc2, bc2)
